# initial kernel scaffold (unmeasured)
import jax
import jax.numpy as jnp
from jax import lax
from jax.experimental import pallas as pl
from jax.experimental.pallas import tpu as pltpu

B, S, H, Dh, Dr = 2, 512, 16, 128, 32
D = 2048
DC_SH = 128
SCALE = (Dh + Dr) ** -0.5

_F32 = jnp.float32


def _dot(a, b):
    return jnp.dot(a, b, preferred_element_type=_F32)


def _kv_exchange(x2d, Wdkv, Wuk, Wuv):
    def body(x_ref, wdkv_ref, wuk_ref, wuv_ref, k_ref, v_ref,
             c_loc, c_oth, wuk_oth, wuv_oth, send_sems, recv_sems):
        my_x = lax.axis_index("x")
        my_y = lax.axis_index("y")
        my_z = lax.axis_index("z")
        partner = (my_x, 1 - my_y, my_z)

        barrier_sem = pltpu.get_barrier_semaphore()
        pl.semaphore_signal(barrier_sem, inc=1, device_id=partner,
                            device_id_type=pl.DeviceIdType.MESH)
        pl.semaphore_wait(barrier_sem, 1)

        rdma_wuk = pltpu.make_async_remote_copy(
            src_ref=wuk_ref, dst_ref=wuk_oth,
            send_sem=send_sems.at[0], recv_sem=recv_sems.at[0],
            device_id=partner, device_id_type=pl.DeviceIdType.MESH)
        rdma_wuk.start()
        rdma_wuv = pltpu.make_async_remote_copy(
            src_ref=wuv_ref, dst_ref=wuv_oth,
            send_sem=send_sems.at[1], recv_sem=recv_sems.at[1],
            device_id=partner, device_id_type=pl.DeviceIdType.MESH)
        rdma_wuv.start()

        c_loc[...] = _dot(x_ref[...], wdkv_ref[...])
        rdma_c = pltpu.make_async_remote_copy(
            src_ref=c_loc, dst_ref=c_oth,
            send_sem=send_sems.at[2], recv_sem=recv_sems.at[2],
            device_id=partner, device_id_type=pl.DeviceIdType.MESH)
        rdma_c.start()

        rdma_wuk.wait()
        rdma_wuv.wait()
        rdma_c.wait()

        k_ref[...] = _dot(c_loc[...], wuk_ref[...]) + _dot(c_oth[...], wuk_oth[...])
        v_ref[...] = _dot(c_loc[...], wuv_ref[...]) + _dot(c_oth[...], wuv_oth[...])

    return pl.pallas_call(
        body,
        out_shape=[jax.ShapeDtypeStruct((B * S, D), _F32)] * 2,
        in_specs=[pl.BlockSpec(memory_space=pltpu.VMEM)] * 4,
        out_specs=[pl.BlockSpec(memory_space=pltpu.VMEM)] * 2,
        scratch_shapes=[
            pltpu.VMEM((B * S, DC_SH), _F32),
            pltpu.VMEM((B * S, DC_SH), _F32),
            pltpu.VMEM((DC_SH, D), _F32),
            pltpu.VMEM((DC_SH, D), _F32),
            pltpu.SemaphoreType.DMA((3,)),
            pltpu.SemaphoreType.DMA((3,)),
        ],
        compiler_params=pltpu.CompilerParams(collective_id=0),
    )(x2d, Wdkv, Wuk, Wuv)


def _projections(x2d, Wq, Wqr, Wkr):
    def body(x_ref, wq_ref, wqr_ref, wkr_ref, q_ref, qr_ref, kr_ref):
        x = x_ref[...]
        q_ref[...] = _dot(x, wq_ref[...])
        qr_ref[...] = _dot(x, wqr_ref[...])
        kr_ref[...] = _dot(x, wkr_ref[...])

    return pl.pallas_call(
        body,
        out_shape=[
            jax.ShapeDtypeStruct((B * S, H * Dh), _F32),
            jax.ShapeDtypeStruct((B * S, H * Dr), _F32),
            jax.ShapeDtypeStruct((B * S, Dr), _F32),
        ],
        in_specs=[pl.BlockSpec(memory_space=pltpu.VMEM)] * 4,
        out_specs=[pl.BlockSpec(memory_space=pltpu.VMEM)] * 3,
    )(x2d, Wq, Wqr, Wkr)


def _attention(q2d, qr2d, kr2d, k2d, v2d):
    def body(q_ref, qr_ref, kr_ref, k_ref, v_ref, o_ref):
        s = lax.dot_general(q_ref[...], k_ref[...],
                            (((1,), (1,)), ((), ())),
                            preferred_element_type=_F32)
        s += lax.dot_general(qr_ref[...], kr_ref[...],
                             (((1,), (1,)), ((), ())),
                             preferred_element_type=_F32)
        s *= SCALE
        m = jnp.max(s, axis=-1, keepdims=True)
        p = jnp.exp(s - m)
        p = p / jnp.sum(p, axis=-1, keepdims=True)
        o_ref[...] = _dot(p, v_ref[...])

    return pl.pallas_call(
        body,
        grid=(B, H),
        in_specs=[
            pl.BlockSpec((S, Dh), lambda b, h: (b, h)),
            pl.BlockSpec((S, Dr), lambda b, h: (b, h)),
            pl.BlockSpec((S, Dr), lambda b, h: (b, 0)),
            pl.BlockSpec((S, Dh), lambda b, h: (b, h)),
            pl.BlockSpec((S, Dh), lambda b, h: (b, h)),
        ],
        out_specs=pl.BlockSpec((S, Dh), lambda b, h: (b, h)),
        out_shape=jax.ShapeDtypeStruct((B * S, H * Dh), _F32),
    )(q2d, qr2d, kr2d, k2d, v2d)


def _out_proj(o2d, Wo):
    def body(o_ref, wo_ref, out_ref):
        out_ref[...] = _dot(o_ref[...], wo_ref[...])

    return pl.pallas_call(
        body,
        out_shape=jax.ShapeDtypeStruct((B * S, D), _F32),
        in_specs=[pl.BlockSpec(memory_space=pltpu.VMEM)] * 2,
        out_specs=pl.BlockSpec(memory_space=pltpu.VMEM),
    )(o2d, Wo)


def kernel(x, Wdkv, Wuk, Wuv, Wq, Wqr, Wkr, Wo):
    x2d = x.reshape(B * S, D)
    k2d, v2d = _kv_exchange(x2d, Wdkv, Wuk, Wuv)
    q2d, qr2d, kr2d = _projections(x2d, Wq, Wqr, Wkr)
    o2d = _attention(q2d, qr2d, kr2d, k2d, v2d)
    out2d = _out_proj(o2d, Wo)
    return out2d.reshape(B, S, D)


# baseline (device time: 114871 ns/iter reference)
import jax
import jax.numpy as jnp
from jax import lax
from jax.experimental import pallas as pl
from jax.experimental.pallas import tpu as pltpu

B, S, H, Dh, Dr = 2, 512, 16, 128, 32
D = 2048
DC_SH = 128
SCALE = (Dh + Dr) ** -0.5

_F32 = jnp.float32


def _dot(a, b):
    return jnp.dot(a, b, preferred_element_type=_F32)


def _kv_exchange(x2d, Wdkv, Wuk, Wuv):
    def body(x_ref, wdkv_ref, wuk_ref, wuv_ref, k_ref, v_ref,
             c_loc, c_oth, wuk_oth, wuv_oth, send_sems, recv_sems):
        my_x = lax.axis_index("x")
        my_y = lax.axis_index("y")
        my_z = lax.axis_index("z")
        partner = (my_x, 1 - my_y, my_z)

        barrier_sem = pltpu.get_barrier_semaphore()
        pl.semaphore_signal(barrier_sem, inc=1, device_id=partner,
                            device_id_type=pl.DeviceIdType.MESH)
        pl.semaphore_wait(barrier_sem, 1)

        rdma_wuk = pltpu.make_async_remote_copy(
            src_ref=wuk_ref, dst_ref=wuk_oth,
            send_sem=send_sems.at[0], recv_sem=recv_sems.at[0],
            device_id=partner, device_id_type=pl.DeviceIdType.MESH)
        rdma_wuk.start()
        rdma_wuv = pltpu.make_async_remote_copy(
            src_ref=wuv_ref, dst_ref=wuv_oth,
            send_sem=send_sems.at[1], recv_sem=recv_sems.at[1],
            device_id=partner, device_id_type=pl.DeviceIdType.MESH)
        rdma_wuv.start()

        c_loc[...] = _dot(x_ref[...], wdkv_ref[...])
        rdma_c = pltpu.make_async_remote_copy(
            src_ref=c_loc, dst_ref=c_oth,
            send_sem=send_sems.at[2], recv_sem=recv_sems.at[2],
            device_id=partner, device_id_type=pl.DeviceIdType.MESH)
        rdma_c.start()

        rdma_wuk.wait()
        rdma_wuv.wait()
        rdma_c.wait()

        k_ref[...] = _dot(c_loc[...], wuk_ref[...]) + _dot(c_oth[...], wuk_oth[...])
        v_ref[...] = _dot(c_loc[...], wuv_ref[...]) + _dot(c_oth[...], wuv_oth[...])

    return pl.pallas_call(
        body,
        out_shape=[jax.ShapeDtypeStruct((B * S, D), _F32)] * 2,
        in_specs=[pl.BlockSpec(memory_space=pltpu.VMEM)] * 4,
        out_specs=[pl.BlockSpec(memory_space=pltpu.VMEM)] * 2,
        scratch_shapes=[
            pltpu.VMEM((B * S, DC_SH), _F32),
            pltpu.VMEM((B * S, DC_SH), _F32),
            pltpu.VMEM((DC_SH, D), _F32),
            pltpu.VMEM((DC_SH, D), _F32),
            pltpu.SemaphoreType.DMA((3,)),
            pltpu.SemaphoreType.DMA((3,)),
        ],
        compiler_params=pltpu.CompilerParams(collective_id=0),
    )(x2d, Wdkv, Wuk, Wuv)


def _projections(x2d, Wq, Wqr, Wkr):
    def body(x_ref, wq_ref, wqr_ref, wkr_ref, q_ref, qr_ref, kr_ref):
        x = x_ref[...]
        q_ref[...] = _dot(x, wq_ref[...])
        qr_ref[...] = _dot(x, wqr_ref[...])
        kr_ref[...] = _dot(x, wkr_ref[...])

    return pl.pallas_call(
        body,
        out_shape=[
            jax.ShapeDtypeStruct((B * S, H * Dh), _F32),
            jax.ShapeDtypeStruct((B * S, H * Dr), _F32),
            jax.ShapeDtypeStruct((B * S, Dr), _F32),
        ],
        in_specs=[pl.BlockSpec(memory_space=pltpu.VMEM)] * 4,
        out_specs=[pl.BlockSpec(memory_space=pltpu.VMEM)] * 3,
    )(x2d, Wq, Wqr, Wkr)


def _attention(q2d, qr2d, kr2d, k2d, v2d):
    def body(q_ref, qr_ref, kr_ref, k_ref, v_ref, o_ref):
        kr = kr_ref[...]
        for h in range(H):
            q = q_ref[:, h * Dh:(h + 1) * Dh]
            k = k_ref[:, h * Dh:(h + 1) * Dh]
            qr = qr_ref[:, h * Dr:(h + 1) * Dr]
            s = lax.dot_general(q, k, (((1,), (1,)), ((), ())),
                                preferred_element_type=_F32)
            s += lax.dot_general(qr, kr, (((1,), (1,)), ((), ())),
                                 preferred_element_type=_F32)
            s *= SCALE
            m = jnp.max(s, axis=-1, keepdims=True)
            p = jnp.exp(s - m)
            p = p / jnp.sum(p, axis=-1, keepdims=True)
            o_ref[:, h * Dh:(h + 1) * Dh] = _dot(p, v_ref[:, h * Dh:(h + 1) * Dh])

    return pl.pallas_call(
        body,
        grid=(B,),
        in_specs=[
            pl.BlockSpec((S, H * Dh), lambda b: (b, 0)),
            pl.BlockSpec((S, H * Dr), lambda b: (b, 0)),
            pl.BlockSpec((S, Dr), lambda b: (b, 0)),
            pl.BlockSpec((S, H * Dh), lambda b: (b, 0)),
            pl.BlockSpec((S, H * Dh), lambda b: (b, 0)),
        ],
        out_specs=pl.BlockSpec((S, H * Dh), lambda b: (b, 0)),
        out_shape=jax.ShapeDtypeStruct((B * S, H * Dh), _F32),
    )(q2d, qr2d, kr2d, k2d, v2d)


def _out_proj(o2d, Wo):
    def body(o_ref, wo_ref, out_ref):
        out_ref[...] = _dot(o_ref[...], wo_ref[...])

    return pl.pallas_call(
        body,
        out_shape=jax.ShapeDtypeStruct((B * S, D), _F32),
        in_specs=[pl.BlockSpec(memory_space=pltpu.VMEM)] * 2,
        out_specs=pl.BlockSpec(memory_space=pltpu.VMEM),
    )(o2d, Wo)


def kernel(x, Wdkv, Wuk, Wuv, Wq, Wqr, Wkr, Wo):
    x2d = x.reshape(B * S, D)
    k2d, v2d = _kv_exchange(x2d, Wdkv, Wuk, Wuv)
    q2d, qr2d, kr2d = _projections(x2d, Wq, Wqr, Wkr)
    o2d = _attention(q2d, qr2d, kr2d, k2d, v2d)
    out2d = _out_proj(o2d, Wo)
    return out2d.reshape(B, S, D)


# device time: 79604 ns/iter; 1.4430x vs baseline; 1.4430x over previous
import jax
import jax.numpy as jnp
from jax import lax
from jax.experimental import pallas as pl
from jax.experimental.pallas import tpu as pltpu

B, S, H, Dh, Dr = 2, 512, 16, 128, 32
D = 2048
DC_SH = 128
SCALE = (Dh + Dr) ** -0.5

_F32 = jnp.float32
_BF16 = jnp.bfloat16


def _dot(a, b):
    return jnp.dot(a, b, preferred_element_type=_F32)


def _dot_t(a, b):
    return lax.dot_general(a, b, (((1,), (1,)), ((), ())),
                           preferred_element_type=_F32)


def _exchange_and_project(x2d, Wdkv, Wuk, Wuv, Wq, Wqr, Wkr):
    def body(x_ref, wdkv_ref, wuk_ref, wuv_ref, wq_ref, wqr_ref, wkr_ref,
             q_ref, qr_ref, kr_ref, k_ref, v_ref,
             c_loc, c_oth, wuk_snd, wuk_oth, wuv_snd, wuv_oth,
             send_sems, recv_sems):
        my_x = lax.axis_index("x")
        my_y = lax.axis_index("y")
        my_z = lax.axis_index("z")
        partner = (my_x, 1 - my_y, my_z)

        barrier_sem = pltpu.get_barrier_semaphore()
        pl.semaphore_signal(barrier_sem, inc=1, device_id=partner,
                            device_id_type=pl.DeviceIdType.MESH)
        pl.semaphore_wait(barrier_sem, 1)

        wuk_snd[...] = wuk_ref[...].astype(_BF16)
        rdma_wuk = pltpu.make_async_remote_copy(
            src_ref=wuk_snd, dst_ref=wuk_oth,
            send_sem=send_sems.at[0], recv_sem=recv_sems.at[0],
            device_id=partner, device_id_type=pl.DeviceIdType.MESH)
        rdma_wuk.start()
        wuv_snd[...] = wuv_ref[...].astype(_BF16)
        rdma_wuv = pltpu.make_async_remote_copy(
            src_ref=wuv_snd, dst_ref=wuv_oth,
            send_sem=send_sems.at[1], recv_sem=recv_sems.at[1],
            device_id=partner, device_id_type=pl.DeviceIdType.MESH)
        rdma_wuv.start()

        c_loc[...] = _dot(x_ref[...], wdkv_ref[...].astype(_BF16)).astype(_BF16)
        rdma_c = pltpu.make_async_remote_copy(
            src_ref=c_loc, dst_ref=c_oth,
            send_sem=send_sems.at[2], recv_sem=recv_sems.at[2],
            device_id=partner, device_id_type=pl.DeviceIdType.MESH)
        rdma_c.start()

        q_ref[...] = _dot(x_ref[...], wq_ref[...].astype(_BF16)).astype(_BF16)
        qr_ref[...] = _dot(x_ref[...], wqr_ref[...].astype(_BF16)).astype(_BF16)
        kr_ref[...] = _dot(x_ref[...], wkr_ref[...].astype(_BF16)).astype(_BF16)

        rdma_wuk.wait()
        rdma_wuv.wait()
        rdma_c.wait()

        k_ref[...] = (_dot(c_loc[...], wuk_snd[...])
                      + _dot(c_oth[...], wuk_oth[...])).astype(_BF16)
        v_ref[...] = (_dot(c_loc[...], wuv_snd[...])
                      + _dot(c_oth[...], wuv_oth[...])).astype(_BF16)

    return pl.pallas_call(
        body,
        out_shape=[
            jax.ShapeDtypeStruct((B * S, H * Dh), _BF16),
            jax.ShapeDtypeStruct((B * S, H * Dr), _BF16),
            jax.ShapeDtypeStruct((B * S, Dr), _BF16),
            jax.ShapeDtypeStruct((B * S, D), _BF16),
            jax.ShapeDtypeStruct((B * S, D), _BF16),
        ],
        in_specs=[pl.BlockSpec(memory_space=pltpu.VMEM)] * 7,
        out_specs=[pl.BlockSpec(memory_space=pltpu.VMEM)] * 5,
        scratch_shapes=[
            pltpu.VMEM((B * S, DC_SH), _BF16),
            pltpu.VMEM((B * S, DC_SH), _BF16),
            pltpu.VMEM((DC_SH, D), _BF16),
            pltpu.VMEM((DC_SH, D), _BF16),
            pltpu.VMEM((DC_SH, D), _BF16),
            pltpu.VMEM((DC_SH, D), _BF16),
            pltpu.SemaphoreType.DMA((3,)),
            pltpu.SemaphoreType.DMA((3,)),
        ],
        compiler_params=pltpu.CompilerParams(collective_id=0),
    )(x2d, Wdkv, Wuk, Wuv, Wq, Wqr, Wkr)


def _attention_out(q2d, qr2d, kr2d, k2d, v2d, Wo):
    def body(q_ref, qr_ref, kr_ref, k_ref, v_ref, wo_ref, out_ref, o_scr):
        kr = kr_ref[...]
        for h in range(H):
            q = q_ref[:, h * Dh:(h + 1) * Dh]
            k = k_ref[:, h * Dh:(h + 1) * Dh]
            qr = qr_ref[:, h * Dr:(h + 1) * Dr]
            s = _dot_t(q, k) + _dot_t(qr, kr)
            s *= SCALE
            m = jnp.max(s, axis=-1, keepdims=True)
            p = jnp.exp(s - m)
            p = (p / jnp.sum(p, axis=-1, keepdims=True)).astype(_BF16)
            o_scr[:, h * Dh:(h + 1) * Dh] = _dot(
                p, v_ref[:, h * Dh:(h + 1) * Dh]).astype(_BF16)
        out_ref[...] = _dot(o_scr[...], wo_ref[...])

    return pl.pallas_call(
        body,
        grid=(B,),
        in_specs=[
            pl.BlockSpec((S, H * Dh), lambda b: (b, 0)),
            pl.BlockSpec((S, H * Dr), lambda b: (b, 0)),
            pl.BlockSpec((S, Dr), lambda b: (b, 0)),
            pl.BlockSpec((S, H * Dh), lambda b: (b, 0)),
            pl.BlockSpec((S, H * Dh), lambda b: (b, 0)),
            pl.BlockSpec(memory_space=pltpu.VMEM),
        ],
        out_specs=pl.BlockSpec((S, D), lambda b: (b, 0)),
        out_shape=jax.ShapeDtypeStruct((B * S, D), _F32),
        scratch_shapes=[
            pltpu.VMEM((S, H * Dh), _BF16),
        ],
    )(q2d, qr2d, kr2d, k2d, v2d, Wo)


def kernel(x, Wdkv, Wuk, Wuv, Wq, Wqr, Wkr, Wo):
    x2d = x.reshape(B * S, D).astype(_BF16)
    q2d, qr2d, kr2d, k2d, v2d = _exchange_and_project(
        x2d, Wdkv, Wuk, Wuv, Wq, Wqr, Wkr)
    out2d = _attention_out(q2d, qr2d, kr2d, k2d, v2d, Wo.astype(_BF16))
    return out2d.reshape(B, S, D)


# device time: 67362 ns/iter; 1.7053x vs baseline; 1.1817x over previous
import jax
import jax.numpy as jnp
from jax import lax
from jax.experimental import pallas as pl
from jax.experimental.pallas import tpu as pltpu

B, S, H, Dh, Dr = 2, 512, 16, 128, 32
D = 2048
DC_SH = 128
SCALE = (Dh + Dr) ** -0.5

_F32 = jnp.float32
_BF16 = jnp.bfloat16


def _dot(a, b):
    return jnp.dot(a, b, preferred_element_type=_F32)


def _dot_t(a, b):
    return lax.dot_general(a, b, (((1,), (1,)), ((), ())),
                           preferred_element_type=_F32)


def _exchange_and_project(x2d, Wdkv, Wuk, Wuv, Wq, Wqr, Wkr):
    def body(x_ref, wdkv_ref, wuk_ref, wuv_ref, wq_ref, wqr_ref, wkr_ref,
             q_ref, qr_ref, kr_ref, k_ref, v_ref,
             c_loc, c_oth, wuk_snd, wuk_oth, wuv_snd, wuv_oth,
             send_sems, recv_sems):
        my_x = lax.axis_index("x")
        my_y = lax.axis_index("y")
        my_z = lax.axis_index("z")
        partner = (my_x, 1 - my_y, my_z)

        barrier_sem = pltpu.get_barrier_semaphore()
        pl.semaphore_signal(barrier_sem, inc=1, device_id=partner,
                            device_id_type=pl.DeviceIdType.MESH)
        pl.semaphore_wait(barrier_sem, 1)

        wuk_snd[...] = wuk_ref[...].astype(_BF16)
        rdma_wuk = pltpu.make_async_remote_copy(
            src_ref=wuk_snd, dst_ref=wuk_oth,
            send_sem=send_sems.at[0], recv_sem=recv_sems.at[0],
            device_id=partner, device_id_type=pl.DeviceIdType.MESH)
        rdma_wuk.start()
        wuv_snd[...] = wuv_ref[...].astype(_BF16)
        rdma_wuv = pltpu.make_async_remote_copy(
            src_ref=wuv_snd, dst_ref=wuv_oth,
            send_sem=send_sems.at[1], recv_sem=recv_sems.at[1],
            device_id=partner, device_id_type=pl.DeviceIdType.MESH)
        rdma_wuv.start()

        c_loc[...] = _dot(x_ref[...], wdkv_ref[...].astype(_BF16)).astype(_BF16)
        rdma_c = pltpu.make_async_remote_copy(
            src_ref=c_loc, dst_ref=c_oth,
            send_sem=send_sems.at[2], recv_sem=recv_sems.at[2],
            device_id=partner, device_id_type=pl.DeviceIdType.MESH)
        rdma_c.start()

        q_ref[...] = (_dot(x_ref[...], wq_ref[...].astype(_BF16))
                      * SCALE).astype(_BF16)
        qr_ref[...] = (_dot(x_ref[...], wqr_ref[...].astype(_BF16))
                       * SCALE).astype(_BF16)
        kr_ref[...] = _dot(x_ref[...], wkr_ref[...].astype(_BF16)).astype(_BF16)

        rdma_wuk.wait()
        rdma_wuv.wait()
        rdma_c.wait()

        k_ref[...] = (_dot(c_loc[...], wuk_snd[...])
                      + _dot(c_oth[...], wuk_oth[...])).astype(_BF16)
        v_ref[...] = (_dot(c_loc[...], wuv_snd[...])
                      + _dot(c_oth[...], wuv_oth[...])).astype(_BF16)

    return pl.pallas_call(
        body,
        out_shape=[
            jax.ShapeDtypeStruct((B * S, H * Dh), _BF16),
            jax.ShapeDtypeStruct((B * S, H * Dr), _BF16),
            jax.ShapeDtypeStruct((B * S, Dr), _BF16),
            jax.ShapeDtypeStruct((B * S, D), _BF16),
            jax.ShapeDtypeStruct((B * S, D), _BF16),
        ],
        in_specs=[pl.BlockSpec(memory_space=pltpu.VMEM)] * 7,
        out_specs=[pl.BlockSpec(memory_space=pltpu.VMEM)] * 5,
        scratch_shapes=[
            pltpu.VMEM((B * S, DC_SH), _BF16),
            pltpu.VMEM((B * S, DC_SH), _BF16),
            pltpu.VMEM((DC_SH, D), _BF16),
            pltpu.VMEM((DC_SH, D), _BF16),
            pltpu.VMEM((DC_SH, D), _BF16),
            pltpu.VMEM((DC_SH, D), _BF16),
            pltpu.SemaphoreType.DMA((3,)),
            pltpu.SemaphoreType.DMA((3,)),
        ],
        compiler_params=pltpu.CompilerParams(collective_id=0),
    )(x2d, Wdkv, Wuk, Wuv, Wq, Wqr, Wkr)


_N_WO_CHUNKS = 4
_WO_ROWS = D // _N_WO_CHUNKS


def _attention_out(q2d, qr2d, kr2d, k2d, v2d, Wo):
    def body(q_ref, qr_ref, kr_ref, k_ref, v_ref, wo_hbm, out_ref,
             o_scr, wo_bf, stg0, stg1, dma_sem):
        b = pl.program_id(0)
        stages = (stg0, stg1)

        def wo_chunk_copy(j):
            return pltpu.make_async_copy(
                wo_hbm.at[pl.ds(j * _WO_ROWS, _WO_ROWS), :],
                stages[j % 2], dma_sem)

        @pl.when(b == 0)
        def _():
            wo_chunk_copy(0).start()

        kr = kr_ref[...]
        for h in range(H):
            if h % 4 == 2:
                j = h // 4

                @pl.when(b == 0)
                def _(j=j):
                    wo_chunk_copy(j).wait()
                    if j + 1 < _N_WO_CHUNKS:
                        wo_chunk_copy(j + 1).start()
                    wo_bf[j * _WO_ROWS:(j + 1) * _WO_ROWS, :] = (
                        stages[j % 2][...].astype(_BF16))

            q = q_ref[:, h * Dh:(h + 1) * Dh]
            k = k_ref[:, h * Dh:(h + 1) * Dh]
            qr = qr_ref[:, h * Dr:(h + 1) * Dr]
            s = _dot_t(q, k) + _dot_t(qr, kr)
            p = jnp.exp(s)
            o_un = _dot(p.astype(_BF16), v_ref[:, h * Dh:(h + 1) * Dh])
            rs = jnp.sum(p, axis=-1, keepdims=True)
            o_scr[:, h * Dh:(h + 1) * Dh] = (o_un * (1.0 / rs)).astype(_BF16)

        out_ref[...] = _dot(o_scr[...], wo_bf[...])

    return pl.pallas_call(
        body,
        grid=(B,),
        in_specs=[
            pl.BlockSpec((S, H * Dh), lambda b: (b, 0)),
            pl.BlockSpec((S, H * Dr), lambda b: (b, 0)),
            pl.BlockSpec((S, Dr), lambda b: (b, 0)),
            pl.BlockSpec((S, H * Dh), lambda b: (b, 0)),
            pl.BlockSpec((S, H * Dh), lambda b: (b, 0)),
            pl.BlockSpec(memory_space=pltpu.MemorySpace.HBM),
        ],
        out_specs=pl.BlockSpec((S, D), lambda b: (b, 0)),
        out_shape=jax.ShapeDtypeStruct((B * S, D), _F32),
        scratch_shapes=[
            pltpu.VMEM((S, H * Dh), _BF16),
            pltpu.VMEM((D, D), _BF16),
            pltpu.VMEM((_WO_ROWS, D), _F32),
            pltpu.VMEM((_WO_ROWS, D), _F32),
            pltpu.SemaphoreType.DMA,
        ],
    )(q2d, qr2d, kr2d, k2d, v2d, Wo)


def kernel(x, Wdkv, Wuk, Wuv, Wq, Wqr, Wkr, Wo):
    x2d = x.reshape(B * S, D).astype(_BF16)
    q2d, qr2d, kr2d, k2d, v2d = _exchange_and_project(
        x2d, Wdkv, Wuk, Wuv, Wq, Wqr, Wkr)
    out2d = _attention_out(q2d, qr2d, kr2d, k2d, v2d, Wo)
    return out2d.reshape(B, S, D)
